# baseline (speedup 1.0000x reference)
import jax
import jax.numpy as jnp
from jax.experimental import pallas as pl
from jax.experimental.pallas import tpu as pltpu
from jax.sharding import PartitionSpec as P

_BS = 64
_N = 3000
_NPAD = 3072
_J_TILE = 1024
_N_JT = _NPAD // _J_TILE
_G = 4
_SYM = (12, 15, 18, 19, 20)
_PADVAL = 1e15


def _loss_kernel(params_ref, mpT_ref, tgtT_ref, tgt_ref, out_ref):
    lane = jax.lax.broadcasted_iota(jnp.int32, (1, _NPAD), 1)
    lvalid = (lane < _N).astype(jnp.float32)

    for g in range(_G):
        mpx = mpT_ref[g, 0:1, :]
        mpy = mpT_ref[g, 1:2, :]
        mpz = mpT_ref[g, 2:3, :]

        def p(k, g=g):
            return params_ref[g, 0, k]

        tfx = mpx * p(0) + mpy * p(3) + mpz * p(6) + p(9)
        tfy = mpx * p(1) + mpy * p(4) + mpz * p(7) + p(10)
        tfz = mpx * p(2) + mpy * p(5) + mpz * p(8) + p(11)

        m = p(12)

        @pl.when(m > 0.5)
        def _sym(g=g, tfx=tfx, tfy=tfy, tfz=tfz):
            bx = -2.0 * tfx
            by = -2.0 * tfy
            bz = -2.0 * tfz
            minacc = jnp.full((1, _NPAD), jnp.inf, dtype=jnp.float32)
            for jt in range(_N_JT):
                tg = tgt_ref[g, jt * _J_TILE:(jt + 1) * _J_TILE, :]
                cx = tg[:, 0:1]
                cy = tg[:, 1:2]
                cz = tg[:, 2:3]
                r2c = cx * cx + cy * cy + cz * cz
                v = (r2c + cx * bx) + (cy * by + cz * bz)
                minacc = jnp.minimum(minacc, jnp.min(v, axis=0, keepdims=True))
            q2 = tfx * tfx + tfy * tfy + tfz * tfz
            d2 = jnp.maximum(minacc + q2, 0.0)
            s = jnp.sum(jnp.sqrt(d2) * lvalid, axis=1, keepdims=True)
            out_ref[g] = s

        @pl.when(m <= 0.5)
        def _plain(g=g, tfx=tfx, tfy=tfy, tfz=tfz):
            dx = tfx - tgtT_ref[g, 0:1, :]
            dy = tfy - tgtT_ref[g, 1:2, :]
            dz = tfz - tgtT_ref[g, 2:3, :]
            d2 = dx * dx + dy * dy + dz * dz
            s = jnp.sum(jnp.sqrt(d2) * lvalid, axis=1, keepdims=True)
            out_ref[g] = s


def _dist_block(params, mpT, tgtT, tgt_p):
    bs_local = params.shape[0]
    return pl.pallas_call(
        _loss_kernel,
        grid=(bs_local // _G,),
        in_specs=[
            pl.BlockSpec((_G, 1, 16), lambda b: (b, 0, 0), memory_space=pltpu.SMEM),
            pl.BlockSpec((_G, 3, _NPAD), lambda b: (b, 0, 0)),
            pl.BlockSpec((_G, 3, _NPAD), lambda b: (b, 0, 0)),
            pl.BlockSpec((_G, _NPAD, 3), lambda b: (b, 0, 0)),
        ],
        out_specs=pl.BlockSpec((_G, 1, 1), lambda b: (b, 0, 0)),
        out_shape=jax.ShapeDtypeStruct((bs_local, 1, 1), jnp.float32),
    )(params, mpT, tgtT, tgt_p)


def kernel(pred_r, pred_t, target, model_points, idx):
    bs, num_p, _ = target.shape

    q = pred_r / jnp.linalg.norm(pred_r, axis=1, keepdims=True)
    w, x, y, z = q[:, 0], q[:, 1], q[:, 2], q[:, 3]
    r00 = 1.0 - 2.0 * (y * y + z * z)
    r01 = 2.0 * (x * y - w * z)
    r02 = 2.0 * (x * z + w * y)
    r10 = 2.0 * (x * y + w * z)
    r11 = 1.0 - 2.0 * (x * x + z * z)
    r12 = 2.0 * (y * z - w * x)
    r20 = 2.0 * (x * z - w * y)
    r21 = 2.0 * (y * z + w * x)
    r22 = 1.0 - 2.0 * (x * x + y * y)
    sym = jnp.asarray(_SYM, dtype=idx.dtype)
    mask = (idx[:, 0][:, None] == sym[None, :]).any(axis=1).astype(jnp.float32)
    zeros = jnp.zeros_like(w)
    params = jnp.stack(
        [r00, r01, r02, r10, r11, r12, r20, r21, r22,
         pred_t[:, 0], pred_t[:, 1], pred_t[:, 2], mask, zeros, zeros, zeros],
        axis=1).reshape(bs, 1, 16)

    pad_n = _NPAD - num_p
    mpT = jnp.pad(jnp.transpose(model_points, (0, 2, 1)),
                  ((0, 0), (0, 0), (0, pad_n)))
    tgtT = jnp.pad(jnp.transpose(target, (0, 2, 1)),
                   ((0, 0), (0, 0), (0, pad_n)), constant_values=_PADVAL)
    tgt_p = jnp.pad(target, ((0, 0), (0, pad_n), (0, 0)),
                    constant_values=_PADVAL)

    out = _dist_block(params, mpT, tgtT, tgt_p)

    return out[:, 0, 0] / jnp.float32(num_p)


# speedup vs baseline: 1.0295x; 1.0295x over previous
import jax
import jax.numpy as jnp
from jax.experimental import pallas as pl
from jax.experimental.pallas import tpu as pltpu
from jax.sharding import PartitionSpec as P

_BS = 64
_N = 3000
_NPAD = 3072
_J_TILE = 1024
_N_JT = _NPAD // _J_TILE
_G = 4
_SYM = (12, 15, 18, 19, 20)
_PADVAL = 1e15


def _loss_kernel(params_ref, mpT_ref, tgtT_ref, tgtP_ref, out_ref):
    lane = jax.lax.broadcasted_iota(jnp.int32, (1, _NPAD), 1)
    lvalid = (lane < _N).astype(jnp.float32)

    for g in range(_G):
        mpx = mpT_ref[g, 0:1, :]
        mpy = mpT_ref[g, 1:2, :]
        mpz = mpT_ref[g, 2:3, :]

        def p(k, g=g):
            return params_ref[g, 0, k]

        tfx = mpx * p(0) + mpy * p(3) + mpz * p(6) + p(9)
        tfy = mpx * p(1) + mpy * p(4) + mpz * p(7) + p(10)
        tfz = mpx * p(2) + mpy * p(5) + mpz * p(8) + p(11)

        m = p(12)

        @pl.when(m > 0.5)
        def _sym(g=g, tfx=tfx, tfy=tfy, tfz=tfz):
            bx = -2.0 * tfx
            by = -2.0 * tfy
            bz = -2.0 * tfz
            xT = jnp.transpose(tgtP_ref[g, 0])
            yT = jnp.transpose(tgtP_ref[g, 1])
            zT = jnp.transpose(tgtP_ref[g, 2])
            r2T = xT * xT + yT * yT + zT * zT
            minacc = jnp.full((1, _NPAD), jnp.inf, dtype=jnp.float32)
            for jt in range(_NPAD // 128):
                cx = xT[:, jt:jt + 1]
                cy = yT[:, jt:jt + 1]
                cz = zT[:, jt:jt + 1]
                r2c = r2T[:, jt:jt + 1]
                v = (r2c + cx * bx) + (cy * by + cz * bz)
                minacc = jnp.minimum(minacc, jnp.min(v, axis=0, keepdims=True))
            q2 = tfx * tfx + tfy * tfy + tfz * tfz
            d2 = jnp.maximum(minacc + q2, 0.0)
            s = jnp.sum(jnp.sqrt(d2) * lvalid, axis=1, keepdims=True)
            out_ref[g] = s

        @pl.when(m <= 0.5)
        def _plain(g=g, tfx=tfx, tfy=tfy, tfz=tfz):
            dx = tfx - tgtT_ref[g, 0:1, :]
            dy = tfy - tgtT_ref[g, 1:2, :]
            dz = tfz - tgtT_ref[g, 2:3, :]
            d2 = dx * dx + dy * dy + dz * dz
            s = jnp.sum(jnp.sqrt(d2) * lvalid, axis=1, keepdims=True)
            out_ref[g] = s


def _dist_block(params, mpT, tgtT, tgtP):
    bs_local = params.shape[0]
    return pl.pallas_call(
        _loss_kernel,
        grid=(bs_local // _G,),
        in_specs=[
            pl.BlockSpec((_G, 1, 16), lambda b: (b, 0, 0), memory_space=pltpu.SMEM),
            pl.BlockSpec((_G, 3, _NPAD), lambda b: (b, 0, 0)),
            pl.BlockSpec((_G, 3, _NPAD), lambda b: (b, 0, 0)),
            pl.BlockSpec((_G, 3, _NPAD // 128, 128), lambda b: (b, 0, 0, 0)),
        ],
        out_specs=pl.BlockSpec((_G, 1, 1), lambda b: (b, 0, 0)),
        out_shape=jax.ShapeDtypeStruct((bs_local, 1, 1), jnp.float32),
    )(params, mpT, tgtT, tgtP)


def kernel(pred_r, pred_t, target, model_points, idx):
    bs, num_p, _ = target.shape

    q = pred_r / jnp.linalg.norm(pred_r, axis=1, keepdims=True)
    w, x, y, z = q[:, 0], q[:, 1], q[:, 2], q[:, 3]
    r00 = 1.0 - 2.0 * (y * y + z * z)
    r01 = 2.0 * (x * y - w * z)
    r02 = 2.0 * (x * z + w * y)
    r10 = 2.0 * (x * y + w * z)
    r11 = 1.0 - 2.0 * (x * x + z * z)
    r12 = 2.0 * (y * z - w * x)
    r20 = 2.0 * (x * z - w * y)
    r21 = 2.0 * (y * z + w * x)
    r22 = 1.0 - 2.0 * (x * x + y * y)
    sym = jnp.asarray(_SYM, dtype=idx.dtype)
    mask = (idx[:, 0][:, None] == sym[None, :]).any(axis=1).astype(jnp.float32)
    zeros = jnp.zeros_like(w)
    params = jnp.stack(
        [r00, r01, r02, r10, r11, r12, r20, r21, r22,
         pred_t[:, 0], pred_t[:, 1], pred_t[:, 2], mask, zeros, zeros, zeros],
        axis=1).reshape(bs, 1, 16)

    pad_n = _NPAD - num_p
    mpT = jnp.pad(jnp.transpose(model_points, (0, 2, 1)),
                  ((0, 0), (0, 0), (0, pad_n)))
    tgtT = jnp.pad(jnp.transpose(target, (0, 2, 1)),
                   ((0, 0), (0, 0), (0, pad_n)), constant_values=_PADVAL)
    tgtP = tgtT.reshape(bs, 3, _NPAD // 128, 128)

    out = _dist_block(params, mpT, tgtT, tgtP)

    return out[:, 0, 0] / jnp.float32(num_p)


# speedup vs baseline: 4.5105x; 4.3814x over previous
import jax
import jax.numpy as jnp
from jax.experimental import pallas as pl
from jax.experimental.pallas import tpu as pltpu
from jax.sharding import PartitionSpec as P

_BS = 64
_N = 3000
_NPAD = 3072
_J_TILE = 1024
_N_JT = _NPAD // _J_TILE
_G = 1
_SYM = (12, 15, 18, 19, 20)
_PADVAL = 1e15


def _loss_kernel(params_ref, mpT_ref, tgtT_ref, tgtP_ref, out_ref):
    lane = jax.lax.broadcasted_iota(jnp.int32, (1, _NPAD), 1)
    lvalid = (lane < _N).astype(jnp.float32)

    for g in range(_G):
        mpx = mpT_ref[g, 0:1, :]
        mpy = mpT_ref[g, 1:2, :]
        mpz = mpT_ref[g, 2:3, :]

        def p(k, g=g):
            return params_ref[g, 0, k]

        tfx = mpx * p(0) + mpy * p(3) + mpz * p(6) + p(9)
        tfy = mpx * p(1) + mpy * p(4) + mpz * p(7) + p(10)
        tfz = mpx * p(2) + mpy * p(5) + mpz * p(8) + p(11)

        m = p(12)

        @pl.when(m > 0.5)
        def _sym(g=g, tfx=tfx, tfy=tfy, tfz=tfz):
            bx = -2.0 * tfx
            by = -2.0 * tfy
            bz = -2.0 * tfz
            xT = jnp.transpose(tgtP_ref[g, 0])
            yT = jnp.transpose(tgtP_ref[g, 1])
            zT = jnp.transpose(tgtP_ref[g, 2])
            r2T = xT * xT + yT * yT + zT * zT
            minacc = jnp.full((1, _NPAD), jnp.inf, dtype=jnp.float32)
            for jt in range(_NPAD // 128):
                cx = xT[:, jt:jt + 1]
                cy = yT[:, jt:jt + 1]
                cz = zT[:, jt:jt + 1]
                r2c = r2T[:, jt:jt + 1]
                v = (r2c + cx * bx) + (cy * by + cz * bz)
                minacc = jnp.minimum(minacc, jnp.min(v, axis=0, keepdims=True))
            q2 = tfx * tfx + tfy * tfy + tfz * tfz
            d2 = jnp.maximum(minacc + q2, 0.0)
            s = jnp.sum(jnp.sqrt(d2) * lvalid, axis=1, keepdims=True)
            out_ref[g] = s

        @pl.when(m <= 0.5)
        def _plain(g=g, tfx=tfx, tfy=tfy, tfz=tfz):
            dx = tfx - tgtT_ref[g, 0:1, :]
            dy = tfy - tgtT_ref[g, 1:2, :]
            dz = tfz - tgtT_ref[g, 2:3, :]
            d2 = dx * dx + dy * dy + dz * dz
            s = jnp.sum(jnp.sqrt(d2) * lvalid, axis=1, keepdims=True)
            out_ref[g] = s


def _dist_block(params, mpT, tgtT, tgtP):
    bs_local = params.shape[0]
    return pl.pallas_call(
        _loss_kernel,
        grid=(bs_local // _G,),
        in_specs=[
            pl.BlockSpec((_G, 1, 16), lambda b: (b, 0, 0), memory_space=pltpu.SMEM),
            pl.BlockSpec((_G, 3, _NPAD), lambda b: (b, 0, 0)),
            pl.BlockSpec((_G, 3, _NPAD), lambda b: (b, 0, 0)),
            pl.BlockSpec((_G, 3, _NPAD // 128, 128), lambda b: (b, 0, 0, 0)),
        ],
        out_specs=pl.BlockSpec((_G, 1, 1), lambda b: (b, 0, 0)),
        out_shape=jax.ShapeDtypeStruct((bs_local, 1, 1), jnp.float32),
    )(params, mpT, tgtT, tgtP)


def kernel(pred_r, pred_t, target, model_points, idx):
    bs, num_p, _ = target.shape

    q = pred_r / jnp.linalg.norm(pred_r, axis=1, keepdims=True)
    w, x, y, z = q[:, 0], q[:, 1], q[:, 2], q[:, 3]
    r00 = 1.0 - 2.0 * (y * y + z * z)
    r01 = 2.0 * (x * y - w * z)
    r02 = 2.0 * (x * z + w * y)
    r10 = 2.0 * (x * y + w * z)
    r11 = 1.0 - 2.0 * (x * x + z * z)
    r12 = 2.0 * (y * z - w * x)
    r20 = 2.0 * (x * z - w * y)
    r21 = 2.0 * (y * z + w * x)
    r22 = 1.0 - 2.0 * (x * x + y * y)
    sym = jnp.asarray(_SYM, dtype=idx.dtype)
    mask = (idx[:, 0][:, None] == sym[None, :]).any(axis=1).astype(jnp.float32)
    zeros = jnp.zeros_like(w)
    params = jnp.stack(
        [r00, r01, r02, r10, r11, r12, r20, r21, r22,
         pred_t[:, 0], pred_t[:, 1], pred_t[:, 2], mask, zeros, zeros, zeros],
        axis=1).reshape(bs, 1, 16)

    pad_n = _NPAD - num_p
    mpT = jnp.pad(jnp.transpose(model_points, (0, 2, 1)),
                  ((0, 0), (0, 0), (0, pad_n)))
    tgtT = jnp.pad(jnp.transpose(target, (0, 2, 1)),
                   ((0, 0), (0, 0), (0, pad_n)), constant_values=_PADVAL)
    tgtP = tgtT.reshape(bs, 3, _NPAD // 128, 128)

    out = _dist_block(params, mpT, tgtT, tgtP)

    return out[:, 0, 0] / jnp.float32(num_p)
